# baseline (speedup 1.0000x reference)
import functools

import jax
import jax.numpy as jnp
from jax import lax
from jax.experimental import pallas as pl
from jax.experimental.pallas import tpu as pltpu
from jax.experimental.pallas import tpu_sc as plsc

DIM = 64
NTAB = 1012
VOCAB_P2 = 1002
BATCH = 16384
SEQ = 50
SEQ1 = SEQ + 1
NC, NS = 2, 16
NW = NC * NS
BPW = BATCH // NW
NVEC = BPW // 16
HCOL = DIM // 2


def _gather_body(tab_hbm, idx_hbm, out_hbm,
                 tab_v, idx_v, rows_a, rows_b,
                 tsem, ia_sem, ib_sem, sa_sem, sb_sem):
    sid = lax.axis_index("s")
    wid = sid * NC + lax.axis_index("c")
    b0 = wid * BPW

    pltpu.async_copy(tab_hbm, tab_v, tsem)
    pltpu.async_copy(idx_hbm.at[pl.ds(b0, BPW)], idx_v.at[pl.ds(0, BPW)], ia_sem)
    pltpu.async_copy(
        idx_hbm.at[pl.ds(BATCH + b0, BPW)], idx_v.at[pl.ds(BPW, BPW)], ib_sem
    )
    pltpu.make_async_copy(tab_hbm, tab_v, tsem).wait()

    def compute_half(ioff, rows_v, h):
        @plsc.parallel_loop(0, BPW, 16, unroll=4)
        def vec(b):
            idx16 = idx_v[pl.ds(ioff + b, 16)]
            g16 = idx16 * DIM + (HCOL * h)
            vals = [plsc.load_gather(tab_v, [g16 + k]) for k in range(HCOL)]
            for k in range(HCOL):
                rows_v[0, k, pl.ds(b, 16)] = vals[k]

    def body(s, carry):
        p = lax.rem(s, 2)
        ioff = p * BPW
        isem = None

        @pl.when(p == 0)
        def _wait_idx_a():
            pltpu.make_async_copy(
                idx_hbm.at[pl.ds(b0, BPW)], idx_v.at[pl.ds(0, BPW)], ia_sem
            ).wait()

        @pl.when(p == 1)
        def _wait_idx_b():
            pltpu.make_async_copy(
                idx_hbm.at[pl.ds(b0, BPW)], idx_v.at[pl.ds(0, BPW)], ib_sem
            ).wait()

        for h, (rows_v, ssem) in enumerate(((rows_a, sa_sem), (rows_b, sb_sem))):
            @pl.when(s >= 1)
            def _wait_store():
                pltpu.make_async_copy(
                    rows_v,
                    out_hbm.at[pl.ds(0, 1), pl.ds(0, HCOL), pl.ds(b0, BPW)],
                    ssem,
                ).wait()

            compute_half(ioff, rows_v, h)
            pltpu.async_copy(
                rows_v,
                out_hbm.at[pl.ds(s, 1), pl.ds(HCOL * h, HCOL), pl.ds(b0, BPW)],
                ssem,
            )

        @pl.when((s + 2 < SEQ1) & (p == 0))
        def _prefetch_idx_a():
            pltpu.async_copy(
                idx_hbm.at[pl.ds((s + 2) * BATCH + b0, BPW)],
                idx_v.at[pl.ds(0, BPW)], ia_sem,
            )

        @pl.when((s + 2 < SEQ1) & (p == 1))
        def _prefetch_idx_b():
            pltpu.async_copy(
                idx_hbm.at[pl.ds((s + 2) * BATCH + b0, BPW)],
                idx_v.at[pl.ds(BPW, BPW)], ib_sem,
            )

        return carry

    lax.fori_loop(0, SEQ1, body, 0)
    pltpu.make_async_copy(
        rows_a, out_hbm.at[pl.ds(0, 1), pl.ds(0, HCOL), pl.ds(b0, BPW)], sa_sem
    ).wait()
    pltpu.make_async_copy(
        rows_b, out_hbm.at[pl.ds(0, 1), pl.ds(0, HCOL), pl.ds(b0, BPW)], sb_sem
    ).wait()


_sc_gather = functools.partial(
    pl.kernel,
    out_type=jax.ShapeDtypeStruct((SEQ1, DIM, BATCH), jnp.float32),
    mesh=plsc.VectorSubcoreMesh(core_axis_name="c", subcore_axis_name="s"),
    scratch_types=[
        pltpu.VMEM((NTAB * DIM,), jnp.float32),
        pltpu.VMEM((2 * BPW,), jnp.int32),
        pltpu.VMEM((1, HCOL, BPW), jnp.float32),
        pltpu.VMEM((1, HCOL, BPW), jnp.float32),
        pltpu.SemaphoreType.DMA,
        pltpu.SemaphoreType.DMA,
        pltpu.SemaphoreType.DMA,
        pltpu.SemaphoreType.DMA,
        pltpu.SemaphoreType.DMA,
    ],
    compiler_params=pltpu.CompilerParams(use_tc_tiling_on_sc=True, needs_layout_passes=False),
)(_gather_body)


def kernel(tokens, charges, aa_table, charge_table):
    aa_table = aa_table.at[0].set(0.0)
    table = jnp.concatenate([aa_table, charge_table], axis=0).reshape(-1)
    cidx = jnp.concatenate(
        [charges.astype(jnp.int32)[:, None] + VOCAB_P2, tokens.astype(jnp.int32)],
        axis=1,
    )
    cidx_t = cidx.T.reshape(-1)
    out_t = _sc_gather(table, cidx_t)
    return out_t.transpose(2, 0, 1)


# speedup vs baseline: 1.2573x; 1.2573x over previous
import functools

import jax
import jax.numpy as jnp
from jax import lax
from jax.experimental import pallas as pl
from jax.experimental.pallas import tpu as pltpu
from jax.experimental.pallas import tpu_sc as plsc

DIM = 64
NTAB = 1012
VOCAB_P2 = 1002
BATCH = 16384
SEQ = 50
SEQ1 = SEQ + 1
NC, NS = 2, 16
NW = NC * NS
SEQ_PER_W = BATCH // NW
NSQ = 8
NGRP = SEQ_PER_W // NSQ


def _gather_body(tab_hbm, idx_hbm, out_hbm,
                 tab_sp, idx_v, rows_v, isem, gsem, ssem):
    sid = lax.axis_index("s")
    wid = sid * NC + lax.axis_index("c")
    base_seq = wid * SEQ_PER_W

    @pl.when(sid == 0)
    def _fill_table():
        pltpu.sync_copy(tab_hbm, tab_sp)

    plsc.subcore_barrier()

    pltpu.sync_copy(idx_hbm.at[pl.ds(base_seq, NSQ)], idx_v.at[0])

    def body(i, carry):
        b = lax.rem(i, 2)
        pb = 1 - b
        seq0 = base_seq + i * NSQ

        @pl.when(i >= 1)
        def _wait_idx():
            pltpu.make_async_copy(
                idx_hbm.at[pl.ds(base_seq, NSQ)], idx_v.at[b], isem
            ).wait()

        gathers = [
            pltpu.async_copy(
                tab_sp.at[idx_v.at[b].at[j]], rows_v.at[b].at[j], gsem
            )
            for j in range(NSQ)
        ]

        @pl.when(i < NGRP - 1)
        def _prefetch_idx():
            pltpu.async_copy(
                idx_hbm.at[pl.ds(seq0 + NSQ, NSQ)], idx_v.at[pb], isem
            )

        @pl.when(i >= 1)
        def _wait_store():
            pltpu.make_async_copy(
                rows_v.at[pb], out_hbm.at[pl.ds(base_seq, NSQ)], ssem
            ).wait()

        for c in gathers:
            c.wait()
        pltpu.async_copy(rows_v.at[b], out_hbm.at[pl.ds(seq0, NSQ)], ssem)
        return carry

    lax.fori_loop(0, NGRP, body, 0)
    lb = (NGRP - 1) % 2
    pltpu.make_async_copy(
        rows_v.at[lb], out_hbm.at[pl.ds(base_seq, NSQ)], ssem
    ).wait()


_sc_gather = functools.partial(
    pl.kernel,
    out_type=jax.ShapeDtypeStruct((BATCH, SEQ1, DIM), jnp.float32),
    mesh=plsc.VectorSubcoreMesh(core_axis_name="c", subcore_axis_name="s"),
    scratch_types=[
        pltpu.VMEM_SHARED((NTAB, DIM), jnp.float32),
        pltpu.VMEM((2, NSQ, SEQ1), jnp.int32),
        pltpu.VMEM((2, NSQ, SEQ1, DIM), jnp.float32),
        pltpu.SemaphoreType.DMA,
        pltpu.SemaphoreType.DMA,
        pltpu.SemaphoreType.DMA,
    ],
    compiler_params=pltpu.CompilerParams(use_tc_tiling_on_sc=False),
)(_gather_body)


def kernel(tokens, charges, aa_table, charge_table):
    aa_table = aa_table.at[0].set(0.0)
    table = jnp.concatenate([aa_table, charge_table], axis=0)
    cidx = jnp.concatenate(
        [charges.astype(jnp.int32)[:, None] + VOCAB_P2, tokens.astype(jnp.int32)],
        axis=1,
    )
    return _sc_gather(table, cidx)
